# baseline (speedup 1.0000x reference)
import functools

import jax
import jax.numpy as jnp
from jax import lax
from jax.experimental import pallas as pl
from jax.experimental.pallas import tpu as pltpu
from jax.experimental.pallas import tpu_sc as plsc

K_TOP = 32
MIX = 0.25
BW = 10.0
NEG = -3.0e38
BIG = 2**30


def _scores_body(h_ref, k_ref, out_ref):
    kb = k_ref[...]
    s = lax.dot_general(h_ref[...], kb, (((1,), (1,)), ((), ())),
                        preferred_element_type=jnp.float32)
    ksq = jnp.sum(kb * kb, axis=1)
    out_ref[...] = 2.0 * s - ksq[None, :]


def _scores(h, db_keys, bn):
    q, d = h.shape
    n = db_keys.shape[0]
    return pl.pallas_call(
        _scores_body,
        grid=(n // bn,),
        in_specs=[
            pl.BlockSpec((q, d), lambda j: (0, 0)),
            pl.BlockSpec((bn, d), lambda j: (j, 0)),
        ],
        out_specs=pl.BlockSpec((q, bn), lambda j: (0, j)),
        out_shape=jax.ShapeDtypeStruct((q, n), jnp.float32),
    )(h, db_keys)


def _sc_midsection(scores, db_values, vocab):
    q, n = scores.shape
    info = plsc.get_sparse_core_info()
    nc, ns = info.num_cores, info.num_subcores
    nw = nc * ns
    rows_per_w = q // nw
    ng = 512
    gsz = n // ng
    mesh = plsc.VectorSubcoreMesh(core_axis_name="c", subcore_axis_name="s")

    @functools.partial(
        pl.kernel,
        mesh=mesh,
        compiler_params=pltpu.CompilerParams(needs_layout_passes=False),
        out_type=jax.ShapeDtypeStruct((q, vocab), jnp.float32),
        scratch_types=[
            pltpu.VMEM((n,), jnp.float32),
            pltpu.VMEM((ng,), jnp.float32),
            pltpu.VMEM((K_TOP,), jnp.float32),
            pltpu.VMEM((K_TOP,), jnp.int32),
            pltpu.VMEM((K_TOP,), jnp.int32),
            pltpu.VMEM((vocab,), jnp.float32),
            pltpu.VMEM((16,), jnp.float32),
            pltpu.VMEM((16,), jnp.int32),
            pltpu.SemaphoreType.DMA,
        ],
    )
    def body(scores_hbm, dbv_hbm, out_hbm, row_v, gm_v, tv_v, ti_v, tok_v,
             ebd_v, bf_f, bf_i, sem):
        wid = lax.axis_index("s") * nc + lax.axis_index("c")
        iota = lax.iota(jnp.int32, 16)
        zeros16 = jnp.zeros((16,), jnp.float32)

        def bfly_f(x, op):
            for k in (1, 2, 4, 8):
                bf_f[...] = x
                x = op(x, plsc.load_gather(bf_f, [jnp.bitwise_xor(iota, k)]))
            return x

        def bfly_i(x, op):
            for k in (1, 2, 4, 8):
                bf_i[...] = x
                x = op(x, plsc.load_gather(bf_i, [jnp.bitwise_xor(iota, k)]))
            return x

        def zero_body(i, _):
            ebd_v[pl.ds(i * 16, 16)] = zeros16
            return 0

        lax.fori_loop(0, vocab // 16, zero_body, 0)

        def do_row(r, _):
            row = wid * rows_per_w + r
            pltpu.sync_copy(scores_hbm.at[row], row_v)

            for v in range(ng // 16):
                def p1(t, acc):
                    base = t * (4 * ng) + v * 16
                    a = jnp.maximum(row_v[pl.ds(base, 16)],
                                    row_v[pl.ds(base + ng, 16)])
                    b = jnp.maximum(row_v[pl.ds(base + 2 * ng, 16)],
                                    row_v[pl.ds(base + 3 * ng, 16)])
                    return jnp.maximum(acc, jnp.maximum(a, b))

                acc = lax.fori_loop(0, gsz // 4, p1,
                                    jnp.full((16,), NEG, jnp.float32))
                gm_v[pl.ds(v * 16, 16)] = acc

            def extract(kk, _):
                m = jnp.full((16,), NEG, jnp.float32)
                gidx = jnp.full((16,), BIG, jnp.int32)
                for v in range(ng // 16):
                    x = gm_v[pl.ds(v * 16, 16)]
                    upd = x > m
                    m = jnp.where(upd, x, m)
                    gidx = jnp.where(upd, v * 16 + iota, gidx)
                gmax = bfly_f(m, jnp.maximum)
                g = bfly_i(jnp.where(m == gmax, gidx, BIG), jnp.minimum)

                pvec = jnp.full((16,), BIG, jnp.int32)
                vals = []
                idxs = []
                for u in range(gsz // 16):
                    idx_u = g + ng * (u * 16 + iota)
                    val_u = plsc.load_gather(row_v, [idx_u])
                    vals.append(val_u)
                    idxs.append(idx_u)
                    pvec = jnp.minimum(pvec,
                                       jnp.where(val_u == gmax, idx_u, BIG))
                estar_v = bfly_i(pvec, jnp.minimum)
                nmv = jnp.full((16,), NEG, jnp.float32)
                for u in range(gsz // 16):
                    nmv = jnp.maximum(
                        nmv, jnp.where(idxs[u] == estar_v, NEG, vals[u]))
                nm = bfly_f(nmv, jnp.maximum)

                lane0 = iota == 0
                kk_v = jnp.full((16,), 0, jnp.int32) + kk
                plsc.store_scatter(row_v, [estar_v],
                                   jnp.full((16,), NEG, jnp.float32),
                                   mask=lane0)
                plsc.store_scatter(gm_v, [g], nm, mask=lane0)
                plsc.store_scatter(tv_v, [kk_v], gmax, mask=lane0)
                plsc.store_scatter(ti_v, [kk_v], estar_v, mask=lane0)
                return 0

            lax.fori_loop(0, K_TOP, extract, 0)

            tv0 = tv_v[pl.ds(0, 16)]
            tv1 = tv_v[pl.ds(16, 16)]
            mx = bfly_f(jnp.maximum(tv0, tv1), jnp.maximum)
            e0 = jnp.exp((tv0 - mx) / BW)
            e1 = jnp.exp((tv1 - mx) / BW)
            scale = MIX / bfly_f(e0 + e1, jnp.add)
            w0 = e0 * scale
            w1 = e1 * scale

            pltpu.async_copy(dbv_hbm.at[ti_v], tok_v, sem).wait()
            t0 = tok_v[pl.ds(0, 16)]
            t1 = tok_v[pl.ds(16, 16)]

            for j in range(16):
                mj = iota == j
                plsc.addupdate_scatter(ebd_v, [t0], w0, mask=mj)
                plsc.addupdate_scatter(ebd_v, [t1], w1, mask=mj)

            pltpu.sync_copy(ebd_v, out_hbm.at[row])

            plsc.store_scatter(ebd_v, [t0], zeros16)
            plsc.store_scatter(ebd_v, [t1], zeros16)
            return 0

        lax.fori_loop(0, rows_per_w, do_row, 0)

    return body(scores, db_values)


def _mix_body(lg_ref, ebd_ref, out_ref):
    lg = lg_ref[...]
    m = jnp.max(lg, axis=-1, keepdims=True)
    e = jnp.exp(lg - m)
    p = e / jnp.sum(e, axis=-1, keepdims=True)
    out_ref[...] = jnp.log((1.0 - MIX) * p + ebd_ref[...])


def _mix(lg, ebd, br):
    q, v = lg.shape
    return pl.pallas_call(
        _mix_body,
        grid=(q // br,),
        in_specs=[
            pl.BlockSpec((br, v), lambda i: (i, 0)),
            pl.BlockSpec((br, v), lambda i: (i, 0)),
        ],
        out_specs=pl.BlockSpec((br, v), lambda i: (i, 0)),
        out_shape=jax.ShapeDtypeStruct((q, v), jnp.float32),
    )(lg, ebd)


def kernel(hidden, logits, db_keys, db_values):
    b, s_len, d = hidden.shape
    vocab = logits.shape[-1]
    q = b * s_len
    h = hidden.reshape(q, d)
    lg = logits.reshape(q, vocab)

    dbv = db_values.astype(jnp.int32)
    n_chunks = 4 if q % 4 == 0 else 1
    qc = q // n_chunks
    outs = []
    for c in range(n_chunks):
        hc = lax.slice(h, (c * qc, 0), ((c + 1) * qc, d))
        lgc = lax.slice(lg, (c * qc, 0), ((c + 1) * qc, vocab))
        scores = _scores(hc, db_keys, 2048)
        ebd = _sc_midsection(scores, dbv, vocab)
        outs.append(_mix(lgc, ebd, 16))
    out = jnp.concatenate(outs, axis=0)
    return out.reshape(b, s_len, vocab)


# speedup vs baseline: 1.2550x; 1.2550x over previous
import functools

import jax
import jax.numpy as jnp
from jax import lax
from jax.experimental import pallas as pl
from jax.experimental.pallas import tpu as pltpu
from jax.experimental.pallas import tpu_sc as plsc

K_TOP = 32
MIX = 0.25
BW = 10.0
NEG = -3.0e38
BIG = 2**30
NG = 512
NCHUNK = 8


def _scores_body(h_ref, k_ref, out_ref):
    kb = k_ref[...]
    s = lax.dot_general(h_ref[...], kb, (((1,), (1,)), ((), ())),
                        preferred_element_type=jnp.float32)
    ksq = jnp.sum(kb * kb, axis=1)
    out_ref[...] = 2.0 * s - ksq[None, :]


def _scores(h, db_keys, bn):
    q, d = h.shape
    n = db_keys.shape[0]
    return pl.pallas_call(
        _scores_body,
        grid=(n // bn,),
        in_specs=[
            pl.BlockSpec((q, d), lambda j: (0, 0)),
            pl.BlockSpec((bn, d), lambda j: (j, 0)),
        ],
        out_specs=pl.BlockSpec((q, bn), lambda j: (0, j)),
        out_shape=jax.ShapeDtypeStruct((q, n), jnp.float32),
    )(h, db_keys)


def _sc_midsection(scores, db_values, vocab):
    q, n = scores.shape
    info = plsc.get_sparse_core_info()
    nc, ns = info.num_cores, info.num_subcores
    nw = nc * ns
    rows_per_w = q // nw
    csz = n // NCHUNK
    tpc = (n // NG) // NCHUNK
    mesh = plsc.VectorSubcoreMesh(core_axis_name="c", subcore_axis_name="s")

    @functools.partial(
        pl.kernel,
        mesh=mesh,
        compiler_params=pltpu.CompilerParams(needs_layout_passes=False),
        out_type=jax.ShapeDtypeStruct((q, vocab), jnp.float32),
        scratch_types=[
            pltpu.VMEM((n,), jnp.float32),
            pltpu.VMEM((NG,), jnp.float32),
            pltpu.VMEM((NG // 16,), jnp.float32),
            pltpu.VMEM((K_TOP,), jnp.float32),
            pltpu.VMEM((K_TOP,), jnp.int32),
            pltpu.VMEM((2 * K_TOP,), jnp.int32),
            pltpu.VMEM((n // NG,), jnp.float32),
            pltpu.VMEM((vocab,), jnp.float32),
            pltpu.SemaphoreType.DMA,
            pltpu.SemaphoreType.DMA,
            pltpu.SemaphoreType.DMA,
        ],
    )
    def body(scores_hbm, dbv_hbm, out_hbm, row_v, gm_v, gm2_v, tv_v, ti_v,
             tok_v, probe_v, ebd_v, sem_in, sem_tok, sem_out):
        wid = lax.axis_index("s") * nc + lax.axis_index("c")
        iota = lax.iota(jnp.int32, 16)
        lane0 = iota == 0
        zeros16 = jnp.zeros((16,), jnp.float32)
        negs16 = jnp.full((16,), NEG, jnp.float32)

        def zero_body(i, _):
            ebd_v[pl.ds(i * 16, 16)] = zeros16
            return 0

        lax.fori_loop(0, vocab // 16, zero_body, 0)
        tok_v[pl.ds(0, 16)] = iota * 0
        tok_v[pl.ds(16, 16)] = iota * 0
        tok_v[pl.ds(32, 16)] = iota * 0
        tok_v[pl.ds(48, 16)] = iota * 0

        def do_row(r, _):
            row = wid * rows_per_w + r
            slot = jnp.bitwise_and(r, 1)

            def issue(c, _):
                pltpu.async_copy(
                    scores_hbm.at[row, pl.ds(c * csz, csz)],
                    row_v.at[pl.ds(c * csz, csz)], sem_in)
                return 0

            lax.fori_loop(0, NCHUNK, issue, 0)

            def chunk_body(c, _):
                pltpu.make_async_copy(
                    scores_hbm.at[row, pl.ds(0, csz)],
                    row_v.at[pl.ds(0, csz)], sem_in).wait()
                first = c == 0
                cbase = c * (tpc * NG)
                for v in range(NG // 16):
                    acc = jnp.where(first, negs16, gm_v[pl.ds(v * 16, 16)])
                    for t in range(tpc):
                        acc = jnp.maximum(
                            acc, row_v[pl.ds(cbase + t * NG + v * 16, 16)])
                    gm_v[pl.ds(v * 16, 16)] = acc
                return 0

            lax.fori_loop(0, NCHUNK, chunk_body, 0)

            for v2 in range(NG // 256):
                m2 = negs16
                for j in range(16):
                    x = gm_v[pl.ds((v2 * 16 + j) * 16, 16)]
                    m2 = jnp.where(iota == j, jnp.max(x), m2)
                gm2_v[pl.ds(v2 * 16, 16)] = m2

            def extract(kk, _):
                m2a = gm2_v[pl.ds(0, 16)]
                m2b = gm2_v[pl.ds(16, 16)]
                hi = jnp.maximum(m2a, m2b)
                gmax = jnp.max(hi)
                va = jnp.where(m2a == gmax, iota, BIG)
                vb = jnp.where(m2b == gmax, iota + 16, BIG)
                vstar = jnp.min(jnp.minimum(va, vb))
                gvec = gm_v[pl.ds(vstar * 16, 16)]
                g = jnp.min(jnp.where(gvec == gmax, vstar * 16 + iota, BIG))

                def probe1(u, pv):
                    idx_u = g + NG * (u * 16 + iota)
                    val_u = plsc.load_gather(row_v, [idx_u])
                    probe_v[pl.ds(u * 16, 16)] = val_u
                    return jnp.minimum(pv,
                                       jnp.where(val_u == gmax, idx_u, BIG))

                pvec = lax.fori_loop(0, n // NG // 16, probe1,
                                     jnp.full((16,), BIG, jnp.int32))
                estar = jnp.min(pvec)
                estar_v = jnp.full((16,), estar, jnp.int32)

                def probe2(u, nv):
                    idx_u = g + NG * (u * 16 + iota)
                    val_u = probe_v[pl.ds(u * 16, 16)]
                    return jnp.maximum(nv,
                                       jnp.where(idx_u == estar, NEG, val_u))

                nmv = lax.fori_loop(0, n // NG // 16, probe2, negs16)
                nm = jnp.max(nmv)

                kk_v = jnp.full((16,), 0, jnp.int32) + kk
                plsc.store_scatter(row_v, [estar_v], negs16, mask=lane0)
                gnew = jnp.where(iota == jnp.bitwise_and(g, 15), nm, gvec)
                gm_v[pl.ds(vstar * 16, 16)] = gnew
                plsc.store_scatter(gm2_v, [jnp.full((16,), vstar, jnp.int32)],
                                   jnp.full((16,), jnp.max(gnew), jnp.float32),
                                   mask=lane0)
                plsc.store_scatter(tv_v, [kk_v],
                                   jnp.full((16,), gmax, jnp.float32),
                                   mask=lane0)
                plsc.store_scatter(ti_v, [kk_v], estar_v, mask=lane0)
                return 0

            lax.fori_loop(0, K_TOP, extract, 0)

            @pl.when(r >= 1)
            def _():
                pltpu.make_async_copy(out_hbm.at[row], ebd_v, sem_out).wait()

            sprev = 1 - slot
            old0 = tok_v[pl.ds(sprev * K_TOP, 16)]
            old1 = tok_v[pl.ds(sprev * K_TOP + 16, 16)]
            plsc.store_scatter(ebd_v, [old0], zeros16)
            plsc.store_scatter(ebd_v, [old1], zeros16)

            tokcp = pltpu.async_copy(
                dbv_hbm.at[ti_v], tok_v.at[pl.ds(slot * K_TOP, K_TOP)],
                sem_tok)

            tv0 = tv_v[pl.ds(0, 16)]
            tv1 = tv_v[pl.ds(16, 16)]
            mx = jnp.max(jnp.maximum(tv0, tv1))
            e0 = jnp.exp((tv0 - mx) / BW)
            e1 = jnp.exp((tv1 - mx) / BW)
            scale = MIX / (zeros16 + jnp.sum(e0 + e1))
            w0 = e0 * scale
            w1 = e1 * scale

            tokcp.wait()
            t0 = tok_v[pl.ds(slot * K_TOP, 16)]
            t1 = tok_v[pl.ds(slot * K_TOP + 16, 16)]

            for j in range(16):
                mj = iota == j
                plsc.addupdate_scatter(ebd_v, [t0], w0, mask=mj)
                plsc.addupdate_scatter(ebd_v, [t1], w1, mask=mj)

            pltpu.async_copy(ebd_v, out_hbm.at[row], sem_out)
            return 0

        lax.fori_loop(0, rows_per_w, do_row, 0)

        pltpu.make_async_copy(out_hbm.at[0], ebd_v, sem_out).wait()

    return body(scores, db_values)


def _mix_body(lg_ref, ebd_ref, out_ref):
    lg = lg_ref[...]
    m = jnp.max(lg, axis=-1, keepdims=True)
    e = jnp.exp(lg - m)
    p = e / jnp.sum(e, axis=-1, keepdims=True)
    out_ref[...] = jnp.log((1.0 - MIX) * p + ebd_ref[...])


def _mix(lg, ebd, br):
    q, v = lg.shape
    return pl.pallas_call(
        _mix_body,
        grid=(q // br,),
        in_specs=[
            pl.BlockSpec((br, v), lambda i: (i, 0)),
            pl.BlockSpec((br, v), lambda i: (i, 0)),
        ],
        out_specs=pl.BlockSpec((br, v), lambda i: (i, 0)),
        out_shape=jax.ShapeDtypeStruct((q, v), jnp.float32),
    )(lg, ebd)


def kernel(hidden, logits, db_keys, db_values):
    b, s_len, d = hidden.shape
    vocab = logits.shape[-1]
    q = b * s_len
    h = hidden.reshape(q, d)
    lg = logits.reshape(q, vocab)

    scores = _scores(h, db_keys, 2048)
    ebd = _sc_midsection(scores, db_values.astype(jnp.int32), vocab)
    out = _mix(lg, ebd, 16)
    return out.reshape(b, s_len, vocab)
